# baseline (speedup 1.0000x reference)
import jax
import jax.numpy as jnp
import numpy as np
from jax.experimental import pallas as pl

_BLOCK = 2000
_IN_W = 78
_OUT_W = 64
_NPOS = 128
_P_ROWS = 152

_ELEMENTS = (5, 6, 7, 8, 14, 15, 16)
_RINGSIZE_VALS = (0, 3, 4, 5, 6, 7, 8)

_P_BLOCKS = (
    ("el", 7, 0, 0), ("deg", 7, 4, 0), ("ring", 2, 8, 0), ("chg", 8, 12, 0),
    ("aro", 2, 16, 0), ("hyb", 6, 20, 0), ("hyd", 5, 24, 0),
    ("func", 36, 28, None), ("don", 2, 32, 0), ("acc", 2, 34, 0),
    ("rsz", 7, 36, 6), ("arn", 5, 40, 0), ("fus", 8, 44, 0),
    ("bond", 48, 48, None),
)


def _p_row_starts():
    starts, r = {}, 0
    for name, n, _c, _b in _P_BLOCKS:
        starts[name] = r
        r += n
    assert r == 145
    return starts


_P_STARTS = _p_row_starts()


def _positions():
    plan = []

    def clip_table(col, hi, name):
        for r in range(1, hi + 1):
            plan.append((col, r - 0.5, name, r, r - 1))

    for i, zv in enumerate(_ELEMENTS):
        if i:
            plan.append((0, zv - 0.5, "el", i, 0))
            plan.append((0, zv + 0.5, "el_neg", i, 0))
    clip_table(1, 6, "deg")
    plan.append((5, -0.5, "ring", 1, 0))
    clip_table(2, 7, "chg")
    clip_table(4, 1, "aro")
    clip_table(3, 5, "hyb")
    clip_table(6, 4, "hyd")
    for j in range(18):
        plan.append((7 + j, 0.5, f"func{j}", 1, 0))
    clip_table(25, 1, "don")
    clip_table(26, 1, "acc")
    for i, v in enumerate(_RINGSIZE_VALS):
        if i != 6:
            plan.append((27, v - 0.5, "rsz", i, 6))
            plan.append((27, v + 0.5, "rsz_neg", i, 6))
    clip_table(28, 4, "arn")
    clip_table(29, 7, "fus")
    assert len(plan) <= _NPOS, len(plan)
    return plan


_PLAN = _positions()


def _static_constants():
    sel_np = np.zeros((_IN_W, _NPOS), dtype=np.float32)
    lo_np = np.full((1, _NPOS), 1e30, dtype=np.float32)
    d_np = np.zeros((_NPOS, _P_ROWS), dtype=np.float32)
    for p, (col, thr, name, r_hi, r_lo) in enumerate(_PLAN):
        sel_np[col, p] = 1.0
        lo_np[0, p] = thr
        sign = -1.0 if name.endswith("_neg") else 1.0
        base = name[:-4] if name.endswith("_neg") else name
        if base.startswith("func"):
            j = int(base[4:])
            d_np[p, _P_STARTS["func"] + 2 * j + r_hi] += sign
            d_np[p, _P_STARTS["func"] + 2 * j + r_lo] -= sign
        else:
            d_np[p, _P_STARTS[base] + r_hi] += sign
            d_np[p, _P_STARTS[base] + r_lo] -= sign
    b0_np = np.zeros((1, _P_ROWS), dtype=np.float32)
    for name, _n, _c, base_row in _P_BLOCKS:
        if name == "func":
            for j in range(18):
                b0_np[0, _P_STARTS["func"] + 2 * j] = 1.0
        elif base_row is not None:
            b0_np[0, _P_STARTS[name] + base_row] = 1.0
    return sel_np, lo_np, d_np, b0_np


_SEL_NP, _LO_NP, _D_NP, _B0_NP = _static_constants()


def _body(a_ref, sel_ref, lo_ref, d_ref, b0_ref, extra_ref,
          el_ref, deg_ref, ring_ref, chg_ref, aro_ref, hyb_ref, hyd_ref,
          fused_ref, don_ref, acc_ref, rsz_ref, arn_ref, fus_ref, bond_ref,
          o_ref, p_s, wd_s, wa_s, bias_s):
    i = pl.program_id(0)

    @pl.when(i == 0)
    def _prep():
        p_s[...] = jnp.zeros_like(p_s)
        tables = (
            (el_ref, "el"), (deg_ref, "deg"), (ring_ref, "ring"),
            (chg_ref, "chg"), (aro_ref, "aro"), (hyb_ref, "hyb"),
            (hyd_ref, "hyd"), (fused_ref, "func"), (don_ref, "don"),
            (acc_ref, "acc"), (rsz_ref, "rsz"), (arn_ref, "arn"),
            (fus_ref, "fus"), (bond_ref, "bond"),
        )
        for ref, name in tables:
            r0 = _P_STARTS[name]
            c0 = dict((n, c) for n, _r, c, _b in _P_BLOCKS)[name]
            rows, d = ref.shape
            p_s[r0:r0 + rows, c0:c0 + d] = ref[...]
        wd_s[...] = jax.lax.dot(d_ref[...], p_s[...],
                                preferred_element_type=jnp.float32)
        wa_s[...] = jnp.zeros_like(wa_s)
        wa_s[30:78, 48:64] = bond_ref[...]
        bias_s[...] = jax.lax.dot(b0_ref[...], p_s[...],
                                  preferred_element_type=jnp.float32) \
            + extra_ref[...]

    a = a_ref[...]
    bcast = jax.lax.dot(a, sel_ref[...],
                        preferred_element_type=jnp.float32)
    step = jnp.where(bcast >= lo_ref[...], 1.0, 0.0)
    out = jax.lax.dot(step, wd_s[...],
                      preferred_element_type=jnp.float32)
    out = out + jax.lax.dot(a, wa_s[0:_IN_W, :],
                            preferred_element_type=jnp.float32)
    o_ref[...] = out + bias_s[...]


def kernel(atom_inputs, element_embed, degree_embed, ring_embed, charge_embed,
           aromatic_embed, hybrid_embed, hydrogen_embed, func_embeds,
           h_don_embed, h_acc_embed, ringsize_embed, aroma_num_embed,
           fused_if_embed, func_reduce_w, func_reduce_b, bond_env_w,
           bond_env_b):
    fused36 = jnp.einsum(
        "jrb,jbd->jrd", func_embeds,
        func_reduce_w.reshape(18, 2, 4)).reshape(36, 4)
    extra = jnp.concatenate([
        jnp.zeros((28,), jnp.float32), func_reduce_b,
        jnp.zeros((16,), jnp.float32), bond_env_b]).reshape(1, _OUT_W)

    n = atom_inputs.shape[0]
    assert n % _BLOCK == 0
    grid = (n // _BLOCK,)

    def full(shape):
        nd = len(shape)
        return pl.BlockSpec(shape, lambda i: (0,) * nd)

    tbl = full
    return pl.pallas_call(
        _body,
        grid=grid,
        in_specs=[
            pl.BlockSpec((_BLOCK, _IN_W), lambda i: (i, 0)),
            full((_IN_W, _NPOS)),
            full((1, _NPOS)),
            full((_NPOS, _P_ROWS)),
            full((1, _P_ROWS)),
            full((1, _OUT_W)),
            tbl((7, 4)), tbl((7, 4)), tbl((2, 4)), tbl((8, 4)),
            tbl((2, 4)), tbl((6, 4)), tbl((5, 4)), tbl((36, 4)),
            tbl((2, 2)), tbl((2, 2)), tbl((7, 4)), tbl((5, 4)),
            tbl((8, 4)), tbl((48, 16)),
        ],
        out_specs=pl.BlockSpec((_BLOCK, _OUT_W), lambda i: (i, 0)),
        out_shape=jax.ShapeDtypeStruct((n, _OUT_W), jnp.float32),
        scratch_shapes=[
            pltpu_vmem((_P_ROWS, _OUT_W)),
            pltpu_vmem((_NPOS, _OUT_W)),
            pltpu_vmem((80, _OUT_W)),
            pltpu_vmem((1, _OUT_W)),
        ],
    )(atom_inputs, jnp.asarray(_SEL_NP), jnp.asarray(_LO_NP),
      jnp.asarray(_D_NP), jnp.asarray(_B0_NP), extra,
      element_embed, degree_embed, ring_embed, charge_embed, aromatic_embed,
      hybrid_embed, hydrogen_embed, fused36, h_don_embed, h_acc_embed,
      ringsize_embed, aroma_num_embed, fused_if_embed, bond_env_w)


def pltpu_vmem(shape):
    from jax.experimental.pallas import tpu as pltpu
    return pltpu.VMEM(shape, jnp.float32)


# speedup vs baseline: 1.1781x; 1.1781x over previous
import jax
import jax.numpy as jnp
import numpy as np
from jax.experimental import pallas as pl

_BLOCK = 10000
_IN_W = 78
_OUT_W = 64
_NPOS = 128
_P_ROWS = 152

_ELEMENTS = (5, 6, 7, 8, 14, 15, 16)
_RINGSIZE_VALS = (0, 3, 4, 5, 6, 7, 8)

_P_BLOCKS = (
    ("el", 7, 0, 0), ("deg", 7, 4, 0), ("ring", 2, 8, 0), ("chg", 8, 12, 0),
    ("aro", 2, 16, 0), ("hyb", 6, 20, 0), ("hyd", 5, 24, 0),
    ("func", 36, 28, None), ("don", 2, 32, 0), ("acc", 2, 34, 0),
    ("rsz", 7, 36, 6), ("arn", 5, 40, 0), ("fus", 8, 44, 0),
    ("bond", 48, 48, None),
)


def _p_row_starts():
    starts, r = {}, 0
    for name, n, _c, _b in _P_BLOCKS:
        starts[name] = r
        r += n
    assert r == 145
    return starts


_P_STARTS = _p_row_starts()


def _positions():
    plan = []

    def clip_table(col, hi, name):
        for r in range(1, hi + 1):
            plan.append((col, r - 0.5, name, r, r - 1))

    for i, zv in enumerate(_ELEMENTS):
        if i:
            plan.append((0, zv - 0.5, "el", i, 0))
            plan.append((0, zv + 0.5, "el_neg", i, 0))
    clip_table(1, 6, "deg")
    plan.append((5, -0.5, "ring", 1, 0))
    clip_table(2, 7, "chg")
    clip_table(4, 1, "aro")
    clip_table(3, 5, "hyb")
    clip_table(6, 4, "hyd")
    for j in range(18):
        plan.append((7 + j, 0.5, f"func{j}", 1, 0))
    clip_table(25, 1, "don")
    clip_table(26, 1, "acc")
    for i, v in enumerate(_RINGSIZE_VALS):
        if i != 6:
            plan.append((27, v - 0.5, "rsz", i, 6))
            plan.append((27, v + 0.5, "rsz_neg", i, 6))
    clip_table(28, 4, "arn")
    clip_table(29, 7, "fus")
    assert len(plan) <= _NPOS, len(plan)
    return plan


_PLAN = _positions()


def _static_constants():
    sel_np = np.zeros((_IN_W, _NPOS), dtype=np.float32)
    lo_np = np.full((1, _NPOS), 1e30, dtype=np.float32)
    d_np = np.zeros((_NPOS, _P_ROWS), dtype=np.float32)
    for p, (col, thr, name, r_hi, r_lo) in enumerate(_PLAN):
        sel_np[col, p] = 1.0
        lo_np[0, p] = thr
        sign = -1.0 if name.endswith("_neg") else 1.0
        base = name[:-4] if name.endswith("_neg") else name
        if base.startswith("func"):
            j = int(base[4:])
            d_np[p, _P_STARTS["func"] + 2 * j + r_hi] += sign
            d_np[p, _P_STARTS["func"] + 2 * j + r_lo] -= sign
        else:
            d_np[p, _P_STARTS[base] + r_hi] += sign
            d_np[p, _P_STARTS[base] + r_lo] -= sign
    b0_np = np.zeros((1, _P_ROWS), dtype=np.float32)
    for name, _n, _c, base_row in _P_BLOCKS:
        if name == "func":
            for j in range(18):
                b0_np[0, _P_STARTS["func"] + 2 * j] = 1.0
        elif base_row is not None:
            b0_np[0, _P_STARTS[name] + base_row] = 1.0
    return sel_np, lo_np, d_np, b0_np


_SEL_NP, _LO_NP, _D_NP, _B0_NP = _static_constants()


def _body(a_ref, sel_ref, lo_ref, d_ref, b0_ref, extra_ref,
          el_ref, deg_ref, ring_ref, chg_ref, aro_ref, hyb_ref, hyd_ref,
          fused_ref, don_ref, acc_ref, rsz_ref, arn_ref, fus_ref, bond_ref,
          o_ref, p_s, wd_s, wa_s, bias_s):
    i = pl.program_id(0)

    @pl.when(i == 0)
    def _prep():
        p_s[...] = jnp.zeros_like(p_s)
        tables = (
            (el_ref, "el"), (deg_ref, "deg"), (ring_ref, "ring"),
            (chg_ref, "chg"), (aro_ref, "aro"), (hyb_ref, "hyb"),
            (hyd_ref, "hyd"), (fused_ref, "func"), (don_ref, "don"),
            (acc_ref, "acc"), (rsz_ref, "rsz"), (arn_ref, "arn"),
            (fus_ref, "fus"), (bond_ref, "bond"),
        )
        for ref, name in tables:
            r0 = _P_STARTS[name]
            c0 = dict((n, c) for n, _r, c, _b in _P_BLOCKS)[name]
            rows, d = ref.shape
            p_s[r0:r0 + rows, c0:c0 + d] = ref[...]
        wd_s[...] = jax.lax.dot(d_ref[...], p_s[...],
                                preferred_element_type=jnp.float32)
        wa_s[...] = jnp.zeros_like(wa_s)
        wa_s[30:78, 48:64] = bond_ref[...]
        bias_s[...] = jax.lax.dot(b0_ref[...], p_s[...],
                                  preferred_element_type=jnp.float32) \
            + extra_ref[...]

    a = a_ref[...]
    bcast = jax.lax.dot(a, sel_ref[...],
                        preferred_element_type=jnp.float32)
    step = jnp.where(bcast >= lo_ref[...], 1.0, 0.0)
    out = jax.lax.dot(step, wd_s[...],
                      preferred_element_type=jnp.float32)
    out = out + jax.lax.dot(a, wa_s[0:_IN_W, :],
                            preferred_element_type=jnp.float32)
    o_ref[...] = out + bias_s[...]


def kernel(atom_inputs, element_embed, degree_embed, ring_embed, charge_embed,
           aromatic_embed, hybrid_embed, hydrogen_embed, func_embeds,
           h_don_embed, h_acc_embed, ringsize_embed, aroma_num_embed,
           fused_if_embed, func_reduce_w, func_reduce_b, bond_env_w,
           bond_env_b):
    fused36 = jnp.einsum(
        "jrb,jbd->jrd", func_embeds,
        func_reduce_w.reshape(18, 2, 4)).reshape(36, 4)
    extra = jnp.concatenate([
        jnp.zeros((28,), jnp.float32), func_reduce_b,
        jnp.zeros((16,), jnp.float32), bond_env_b]).reshape(1, _OUT_W)

    n = atom_inputs.shape[0]
    assert n % _BLOCK == 0
    grid = (n // _BLOCK,)

    def full(shape):
        nd = len(shape)
        return pl.BlockSpec(shape, lambda i: (0,) * nd)

    tbl = full
    return pl.pallas_call(
        _body,
        grid=grid,
        in_specs=[
            pl.BlockSpec((_BLOCK, _IN_W), lambda i: (i, 0)),
            full((_IN_W, _NPOS)),
            full((1, _NPOS)),
            full((_NPOS, _P_ROWS)),
            full((1, _P_ROWS)),
            full((1, _OUT_W)),
            tbl((7, 4)), tbl((7, 4)), tbl((2, 4)), tbl((8, 4)),
            tbl((2, 4)), tbl((6, 4)), tbl((5, 4)), tbl((36, 4)),
            tbl((2, 2)), tbl((2, 2)), tbl((7, 4)), tbl((5, 4)),
            tbl((8, 4)), tbl((48, 16)),
        ],
        out_specs=pl.BlockSpec((_BLOCK, _OUT_W), lambda i: (i, 0)),
        out_shape=jax.ShapeDtypeStruct((n, _OUT_W), jnp.float32),
        scratch_shapes=[
            pltpu_vmem((_P_ROWS, _OUT_W)),
            pltpu_vmem((_NPOS, _OUT_W)),
            pltpu_vmem((80, _OUT_W)),
            pltpu_vmem((1, _OUT_W)),
        ],
    )(atom_inputs, jnp.asarray(_SEL_NP), jnp.asarray(_LO_NP),
      jnp.asarray(_D_NP), jnp.asarray(_B0_NP), extra,
      element_embed, degree_embed, ring_embed, charge_embed, aromatic_embed,
      hybrid_embed, hydrogen_embed, fused36, h_don_embed, h_acc_embed,
      ringsize_embed, aroma_num_embed, fused_if_embed, bond_env_w)


def pltpu_vmem(shape):
    from jax.experimental.pallas import tpu as pltpu
    return pltpu.VMEM(shape, jnp.float32)


# speedup vs baseline: 1.1841x; 1.0051x over previous
import jax
import jax.numpy as jnp
import numpy as np
from jax.experimental import pallas as pl

_BLOCK = 20000
_IN_W = 78
_OUT_W = 64
_NPOS = 128
_P_ROWS = 152

_ELEMENTS = (5, 6, 7, 8, 14, 15, 16)
_RINGSIZE_VALS = (0, 3, 4, 5, 6, 7, 8)

_P_BLOCKS = (
    ("el", 7, 0, 0), ("deg", 7, 4, 0), ("ring", 2, 8, 0), ("chg", 8, 12, 0),
    ("aro", 2, 16, 0), ("hyb", 6, 20, 0), ("hyd", 5, 24, 0),
    ("func", 36, 28, None), ("don", 2, 32, 0), ("acc", 2, 34, 0),
    ("rsz", 7, 36, 6), ("arn", 5, 40, 0), ("fus", 8, 44, 0),
    ("bond", 48, 48, None),
)


def _p_row_starts():
    starts, r = {}, 0
    for name, n, _c, _b in _P_BLOCKS:
        starts[name] = r
        r += n
    assert r == 145
    return starts


_P_STARTS = _p_row_starts()


def _positions():
    plan = []

    def clip_table(col, hi, name):
        for r in range(1, hi + 1):
            plan.append((col, r - 0.5, name, r, r - 1))

    for i, zv in enumerate(_ELEMENTS):
        if i:
            plan.append((0, zv - 0.5, "el", i, 0))
            plan.append((0, zv + 0.5, "el_neg", i, 0))
    clip_table(1, 6, "deg")
    plan.append((5, -0.5, "ring", 1, 0))
    clip_table(2, 7, "chg")
    clip_table(4, 1, "aro")
    clip_table(3, 5, "hyb")
    clip_table(6, 4, "hyd")
    for j in range(18):
        plan.append((7 + j, 0.5, f"func{j}", 1, 0))
    clip_table(25, 1, "don")
    clip_table(26, 1, "acc")
    for i, v in enumerate(_RINGSIZE_VALS):
        if i != 6:
            plan.append((27, v - 0.5, "rsz", i, 6))
            plan.append((27, v + 0.5, "rsz_neg", i, 6))
    clip_table(28, 4, "arn")
    clip_table(29, 7, "fus")
    assert len(plan) <= _NPOS, len(plan)
    return plan


_PLAN = _positions()


def _static_constants():
    sel_np = np.zeros((_IN_W, _NPOS), dtype=np.float32)
    lo_np = np.full((1, _NPOS), 1e30, dtype=np.float32)
    d_np = np.zeros((_NPOS, _P_ROWS), dtype=np.float32)
    for p, (col, thr, name, r_hi, r_lo) in enumerate(_PLAN):
        sel_np[col, p] = 1.0
        lo_np[0, p] = thr
        sign = -1.0 if name.endswith("_neg") else 1.0
        base = name[:-4] if name.endswith("_neg") else name
        if base.startswith("func"):
            j = int(base[4:])
            d_np[p, _P_STARTS["func"] + 2 * j + r_hi] += sign
            d_np[p, _P_STARTS["func"] + 2 * j + r_lo] -= sign
        else:
            d_np[p, _P_STARTS[base] + r_hi] += sign
            d_np[p, _P_STARTS[base] + r_lo] -= sign
    b0_np = np.zeros((1, _P_ROWS), dtype=np.float32)
    for name, _n, _c, base_row in _P_BLOCKS:
        if name == "func":
            for j in range(18):
                b0_np[0, _P_STARTS["func"] + 2 * j] = 1.0
        elif base_row is not None:
            b0_np[0, _P_STARTS[name] + base_row] = 1.0
    return sel_np, lo_np, d_np, b0_np


_SEL_NP, _LO_NP, _D_NP, _B0_NP = _static_constants()


def _body(a_ref, sel_ref, lo_ref, d_ref, b0_ref, extra_ref,
          el_ref, deg_ref, ring_ref, chg_ref, aro_ref, hyb_ref, hyd_ref,
          fused_ref, don_ref, acc_ref, rsz_ref, arn_ref, fus_ref, bond_ref,
          o_ref, p_s, wd_s, wa_s, bias_s):
    i = pl.program_id(0)

    @pl.when(i == 0)
    def _prep():
        p_s[...] = jnp.zeros_like(p_s)
        tables = (
            (el_ref, "el"), (deg_ref, "deg"), (ring_ref, "ring"),
            (chg_ref, "chg"), (aro_ref, "aro"), (hyb_ref, "hyb"),
            (hyd_ref, "hyd"), (fused_ref, "func"), (don_ref, "don"),
            (acc_ref, "acc"), (rsz_ref, "rsz"), (arn_ref, "arn"),
            (fus_ref, "fus"), (bond_ref, "bond"),
        )
        for ref, name in tables:
            r0 = _P_STARTS[name]
            c0 = dict((n, c) for n, _r, c, _b in _P_BLOCKS)[name]
            rows, d = ref.shape
            p_s[r0:r0 + rows, c0:c0 + d] = ref[...]
        wd_s[...] = jax.lax.dot(d_ref[...], p_s[...],
                                preferred_element_type=jnp.float32)
        wa_s[...] = jnp.zeros_like(wa_s)
        wa_s[30:78, 48:64] = bond_ref[...]
        bias_s[...] = jax.lax.dot(b0_ref[...], p_s[...],
                                  preferred_element_type=jnp.float32) \
            + extra_ref[...]

    a = a_ref[...]
    bcast = jax.lax.dot(a, sel_ref[...],
                        preferred_element_type=jnp.float32)
    step = jnp.where(bcast >= lo_ref[...], 1.0, 0.0)
    out = jax.lax.dot(step, wd_s[...],
                      preferred_element_type=jnp.float32)
    out = out + jax.lax.dot(a, wa_s[0:_IN_W, :],
                            preferred_element_type=jnp.float32)
    o_ref[...] = out + bias_s[...]


def kernel(atom_inputs, element_embed, degree_embed, ring_embed, charge_embed,
           aromatic_embed, hybrid_embed, hydrogen_embed, func_embeds,
           h_don_embed, h_acc_embed, ringsize_embed, aroma_num_embed,
           fused_if_embed, func_reduce_w, func_reduce_b, bond_env_w,
           bond_env_b):
    fused36 = jnp.einsum(
        "jrb,jbd->jrd", func_embeds,
        func_reduce_w.reshape(18, 2, 4)).reshape(36, 4)
    extra = jnp.concatenate([
        jnp.zeros((28,), jnp.float32), func_reduce_b,
        jnp.zeros((16,), jnp.float32), bond_env_b]).reshape(1, _OUT_W)

    n = atom_inputs.shape[0]
    assert n % _BLOCK == 0
    grid = (n // _BLOCK,)

    def full(shape):
        nd = len(shape)
        return pl.BlockSpec(shape, lambda i: (0,) * nd)

    tbl = full
    return pl.pallas_call(
        _body,
        grid=grid,
        in_specs=[
            pl.BlockSpec((_BLOCK, _IN_W), lambda i: (i, 0)),
            full((_IN_W, _NPOS)),
            full((1, _NPOS)),
            full((_NPOS, _P_ROWS)),
            full((1, _P_ROWS)),
            full((1, _OUT_W)),
            tbl((7, 4)), tbl((7, 4)), tbl((2, 4)), tbl((8, 4)),
            tbl((2, 4)), tbl((6, 4)), tbl((5, 4)), tbl((36, 4)),
            tbl((2, 2)), tbl((2, 2)), tbl((7, 4)), tbl((5, 4)),
            tbl((8, 4)), tbl((48, 16)),
        ],
        out_specs=pl.BlockSpec((_BLOCK, _OUT_W), lambda i: (i, 0)),
        out_shape=jax.ShapeDtypeStruct((n, _OUT_W), jnp.float32),
        scratch_shapes=[
            pltpu_vmem((_P_ROWS, _OUT_W)),
            pltpu_vmem((_NPOS, _OUT_W)),
            pltpu_vmem((80, _OUT_W)),
            pltpu_vmem((1, _OUT_W)),
        ],
    )(atom_inputs, jnp.asarray(_SEL_NP), jnp.asarray(_LO_NP),
      jnp.asarray(_D_NP), jnp.asarray(_B0_NP), extra,
      element_embed, degree_embed, ring_embed, charge_embed, aromatic_embed,
      hybrid_embed, hydrogen_embed, fused36, h_don_embed, h_acc_embed,
      ringsize_embed, aroma_num_embed, fused_if_embed, bond_env_w)


def pltpu_vmem(shape):
    from jax.experimental.pallas import tpu as pltpu
    return pltpu.VMEM(shape, jnp.float32)
